# baseline (speedup 1.0000x reference)
import functools

import jax
import jax.numpy as jnp
from jax import lax
from jax.experimental import pallas as pl
from jax.experimental.pallas import tpu as pltpu
from jax.experimental.pallas import tpu_sc as plsc

B, C, N = 4, 20, 65536
K = 2048
CH = 8192
NV = CH // 16
NCHUNK = N // CH
UNROLL = 8
NKV = K // 16
NPAIR = B * C
NC, NS = 2, 16
NW = NC * NS
JMAX = (NPAIR + NW - 1) // NW


def _splat0(x):
    idx = jnp.zeros((16, 1), jnp.int32)
    return lax.gather(
        x, idx,
        lax.GatherDimensionNumbers(
            offset_dims=(), collapsed_slice_dims=(0,), start_index_map=(0,)),
        (1,), mode=lax.GatherScatterMode.PROMISE_IN_BOUNDS)


def _sc_body(probs_hbm, labels_hbm, res_hbm,
             pbuf0, pbuf1, lbig, hn, hs, hf, stage, sem0, sem1, lsem):
    wid = lax.axis_index("s") * NC + lax.axis_index("c")
    zeros16 = jnp.zeros((16,), jnp.float32)
    ones16 = jnp.ones((16,), jnp.float32)
    pbufs = (pbuf0, pbuf1)
    sems = (sem0, sem1)

    b = wid // 8
    s8 = wid % 8
    pltpu.async_copy(labels_hbm.at[b], lbig, lsem)

    def zero_body(i, _):
        hn[pl.ds(i * 16, 16)] = zeros16
        hs[pl.ds(i * 16, 16)] = zeros16
        hf[pl.ds(i * 16, 16)] = zeros16
        return 0

    lax.fori_loop(0, NKV, zero_body, 0)
    pltpu.make_async_copy(labels_hbm.at[0], lbig, lsem).wait()

    for j in range(JMAX):
        c = s8 + 8 * j

        @pl.when(c < C)
        def _():
            p = b * C + c
            c0 = jnp.where(c == 0, jnp.int32(-1), c)

            def start_load(ci, slot):
                pltpu.async_copy(probs_hbm.at[b, c, pl.ds(ci * CH, CH)],
                                 pbufs[slot], sems[slot])

            def drain(slot):
                pltpu.make_async_copy(probs_hbm.at[0, 0, pl.ds(0, CH)],
                                      pbufs[slot], sems[slot]).wait()

            start_load(0, 0)
            for ci in range(NCHUNK):
                slot = ci % 2
                drain(slot)
                if ci + 1 < NCHUNK:
                    start_load(ci + 1, 1 - slot)
                pbuf = pbufs[slot]
                lbase = ci * CH

                def vec_body(vi, _):
                    base = vi * (16 * UNROLL)
                    loaded = []
                    for u in range(UNROLL):
                        o = base + u * 16
                        loaded.append((pbuf[pl.ds(o, 16)],
                                       lbig[pl.ds(lbase + o, 16)]))
                    outs = []
                    for pv, lv in loaded:
                        valid = lv != 0
                        isfg = lv == c0
                        e = jnp.where(isfg, 1.0 - pv, pv)
                        ec = jnp.minimum(e, 0.99999994)
                        bin_ = (ec * K).astype(jnp.int32)
                        outs.append((valid, isfg, e, bin_))
                    for valid, isfg, e, bin_ in outs:
                        plsc.addupdate_scatter(hn, [bin_], ones16, mask=valid)
                        plsc.addupdate_scatter(hs, [bin_], e, mask=valid)
                        plsc.addupdate_scatter(hf, [bin_], ones16, mask=isfg)
                    return 0

                lax.fori_loop(0, NV // UNROLL, vec_body, 0)

            def psum_body(i, acc):
                return acc + hf[pl.ds(i * 16, 16)]

            P = jnp.sum(lax.fori_loop(0, NKV, psum_body, zeros16))

            def scan_body(i, carry):
                Tc, Fc, acc = carry
                kc = NKV - 1 - i
                nv = hn[pl.ds(kc * 16, 16)]
                sv = hs[pl.ds(kc * 16, 16)]
                fv = hf[pl.ds(kc * 16, 16)]
                hn[pl.ds(kc * 16, 16)] = zeros16
                hs[pl.ds(kc * 16, 16)] = zeros16
                hf[pl.ds(kc * 16, 16)] = zeros16
                Tin = jnp.flip(jnp.cumsum(jnp.flip(nv, 0)), 0) + Tc
                Fin = jnp.flip(jnp.cumsum(jnp.flip(fv, 0)), 0) + Fc
                Tex = Tin - nv
                Fex = Fin - fv
                din = jnp.maximum(P + (Tin - Fin), 1.0)
                dex = jnp.maximum(P + (Tex - Fex), 1.0)
                num = (P - Fex) * din - (P - Fin) * dex
                dj = num * (1.0 / (din * dex))
                vbar = sv * (1.0 / jnp.maximum(nv, 1.0))
                acc = acc + vbar * dj
                return (_splat0(Tin), _splat0(Fin), acc)

            _, _, acc = lax.fori_loop(0, NKV, scan_body,
                                      (zeros16, zeros16, zeros16))
            total_c = jnp.sum(acc)

            lane = lax.iota(jnp.int32, 16)
            row = jnp.where(lane == 0, total_c, jnp.where(lane == 1, P, 0.0))
            stage[...] = row.astype(jnp.float32)
            pltpu.sync_copy(stage, res_hbm.at[p])


_sc_kernel = functools.partial(
    pl.kernel,
    mesh=plsc.VectorSubcoreMesh(core_axis_name="c", subcore_axis_name="s"),
    compiler_params=pltpu.CompilerParams(needs_layout_passes=False),
    out_type=jax.ShapeDtypeStruct((NPAIR, 16), jnp.float32),
    scratch_types=[
        pltpu.VMEM((CH,), jnp.float32),
        pltpu.VMEM((CH,), jnp.float32),
        pltpu.VMEM((N,), jnp.int32),
        pltpu.VMEM((K,), jnp.float32),
        pltpu.VMEM((K,), jnp.float32),
        pltpu.VMEM((K,), jnp.float32),
        pltpu.VMEM((16,), jnp.float32),
        pltpu.SemaphoreType.DMA,
        pltpu.SemaphoreType.DMA,
        pltpu.SemaphoreType.DMA,
    ],
)(_sc_body)


def _combine_body(tot_ref, p_ref, out_ref):
    totals = tot_ref[...]
    present = (p_ref[...] > 0).astype(jnp.float32)
    n = jnp.sum(present, axis=1, keepdims=True)
    tsum = jnp.sum(present * totals, axis=1, keepdims=True)
    loss_b = jnp.where(n > 0, tsum / jnp.maximum(n, 1.0), 0.0)
    out_ref[...] = jnp.sum(loss_b, keepdims=True) * (1.0 / B)


def kernel(uv_out, uv_label):
    res = _sc_kernel(uv_out, uv_label)
    totals = res[:, 0].reshape(B, C)
    ps = res[:, 1].reshape(B, C)
    out = pl.pallas_call(
        _combine_body,
        out_shape=jax.ShapeDtypeStruct((1, 1), jnp.float32),
    )(totals, ps)
    return out[0, 0]


# speedup vs baseline: 1.1276x; 1.1276x over previous
import functools

import jax
import jax.numpy as jnp
from jax import lax
from jax.experimental import pallas as pl
from jax.experimental.pallas import tpu as pltpu
from jax.experimental.pallas import tpu_sc as plsc

B, C, N = 4, 20, 65536
K = 2048
CH = 8192
NV = CH // 16
NCHUNK = N // CH
UNROLL = 8
NKV = K // 16
NPAIR = B * C
NC, NS = 2, 16
NW = NC * NS
JMAX = (NPAIR + NW - 1) // NW


def _splat0(x):
    idx = jnp.zeros((16, 1), jnp.int32)
    return lax.gather(
        x, idx,
        lax.GatherDimensionNumbers(
            offset_dims=(), collapsed_slice_dims=(0,), start_index_map=(0,)),
        (1,), mode=lax.GatherScatterMode.PROMISE_IN_BOUNDS)


def _sc_body(probs_hbm, labels_hbm, res_hbm,
             pbuf0, pbuf1, lbig, hn, hf, stage, sem0, sem1, lsem):
    wid = lax.axis_index("s") * NC + lax.axis_index("c")
    zeros16 = jnp.zeros((16,), jnp.float32)
    ones16 = jnp.ones((16,), jnp.float32)
    pbufs = (pbuf0, pbuf1)
    sems = (sem0, sem1)

    b = wid // 8
    s8 = wid % 8
    pltpu.async_copy(labels_hbm.at[b], lbig, lsem)

    def zero_body(i, _):
        hn[pl.ds(i * 16, 16)] = zeros16
        hf[pl.ds(i * 16, 16)] = zeros16
        return 0

    lax.fori_loop(0, NKV, zero_body, 0)
    pltpu.make_async_copy(labels_hbm.at[0], lbig, lsem).wait()

    for j in range(JMAX):
        c = s8 + 8 * j

        @pl.when(c < C)
        def _():
            p = b * C + c
            c0 = jnp.where(c == 0, jnp.int32(-1), c)

            def start_load(ci, slot):
                pltpu.async_copy(probs_hbm.at[b, c, pl.ds(ci * CH, CH)],
                                 pbufs[slot], sems[slot])

            def drain(slot):
                pltpu.make_async_copy(probs_hbm.at[0, 0, pl.ds(0, CH)],
                                      pbufs[slot], sems[slot]).wait()

            start_load(0, 0)
            for ci in range(NCHUNK):
                slot = ci % 2
                drain(slot)
                if ci + 1 < NCHUNK:
                    start_load(ci + 1, 1 - slot)
                pbuf = pbufs[slot]
                lbase = ci * CH

                def vec_body(vi, _):
                    base = vi * (16 * UNROLL)
                    loaded = []
                    for u in range(UNROLL):
                        o = base + u * 16
                        loaded.append((pbuf[pl.ds(o, 16)],
                                       lbig[pl.ds(lbase + o, 16)]))
                    outs = []
                    for pv, lv in loaded:
                        valid = lv != 0
                        isfg = lv == c0
                        e = jnp.where(isfg, 1.0 - pv, pv)
                        ec = jnp.minimum(e, 0.99999994)
                        bin_ = (ec * K).astype(jnp.int32)
                        outs.append((valid, isfg, bin_))
                    for valid, isfg, bin_ in outs:
                        plsc.addupdate_scatter(hn, [bin_], ones16, mask=valid)
                        plsc.addupdate_scatter(hf, [bin_], ones16, mask=isfg)
                    return 0

                lax.fori_loop(0, NV // UNROLL, vec_body, 0)

            def psum_body(i, acc):
                return acc + hf[pl.ds(i * 16, 16)]

            P = jnp.sum(lax.fori_loop(0, NKV, psum_body, zeros16))

            iota_f = lax.iota(jnp.int32, 16).astype(jnp.float32)

            def scan_body(i, carry):
                Tc, Fc, acc = carry
                kc = NKV - 1 - i
                nv = hn[pl.ds(kc * 16, 16)]
                fv = hf[pl.ds(kc * 16, 16)]
                hn[pl.ds(kc * 16, 16)] = zeros16
                hf[pl.ds(kc * 16, 16)] = zeros16
                Tin = jnp.flip(jnp.cumsum(jnp.flip(nv, 0)), 0) + Tc
                Fin = jnp.flip(jnp.cumsum(jnp.flip(fv, 0)), 0) + Fc
                Tex = Tin - nv
                Fex = Fin - fv
                din = jnp.maximum(P + (Tin - Fin), 1.0)
                dex = jnp.maximum(P + (Tex - Fex), 1.0)
                num = (P - Fex) * din - (P - Fin) * dex
                dj = num * (1.0 / (din * dex))
                vbar = (iota_f + (kc * 16 + 0.5).astype(jnp.float32)) * (1.0 / K)
                acc = acc + vbar * dj
                return (_splat0(Tin), _splat0(Fin), acc)

            _, _, acc = lax.fori_loop(0, NKV, scan_body,
                                      (zeros16, zeros16, zeros16))
            total_c = jnp.sum(acc)

            lane = lax.iota(jnp.int32, 16)
            row = jnp.where(lane == 0, total_c, jnp.where(lane == 1, P, 0.0))
            stage[...] = row.astype(jnp.float32)
            pltpu.sync_copy(stage, res_hbm.at[p])


_sc_kernel = functools.partial(
    pl.kernel,
    mesh=plsc.VectorSubcoreMesh(core_axis_name="c", subcore_axis_name="s"),
    compiler_params=pltpu.CompilerParams(needs_layout_passes=False),
    out_type=jax.ShapeDtypeStruct((NPAIR, 16), jnp.float32),
    scratch_types=[
        pltpu.VMEM((CH,), jnp.float32),
        pltpu.VMEM((CH,), jnp.float32),
        pltpu.VMEM((N,), jnp.int32),
        pltpu.VMEM((K,), jnp.float32),
        pltpu.VMEM((K,), jnp.float32),
        pltpu.VMEM((16,), jnp.float32),
        pltpu.SemaphoreType.DMA,
        pltpu.SemaphoreType.DMA,
        pltpu.SemaphoreType.DMA,
    ],
)(_sc_body)


def _combine_body(tot_ref, p_ref, out_ref):
    totals = tot_ref[...]
    present = (p_ref[...] > 0).astype(jnp.float32)
    n = jnp.sum(present, axis=1, keepdims=True)
    tsum = jnp.sum(present * totals, axis=1, keepdims=True)
    loss_b = jnp.where(n > 0, tsum / jnp.maximum(n, 1.0), 0.0)
    out_ref[...] = jnp.sum(loss_b, keepdims=True) * (1.0 / B)


def kernel(uv_out, uv_label):
    res = _sc_kernel(uv_out, uv_label)
    totals = res[:, 0].reshape(B, C)
    ps = res[:, 1].reshape(B, C)
    out = pl.pallas_call(
        _combine_body,
        out_shape=jax.ShapeDtypeStruct((1, 1), jnp.float32),
    )(totals, ps)
    return out[0, 0]


# speedup vs baseline: 1.2026x; 1.0665x over previous
import functools

import jax
import jax.numpy as jnp
from jax import lax
from jax.experimental import pallas as pl
from jax.experimental.pallas import tpu as pltpu
from jax.experimental.pallas import tpu_sc as plsc

B, C, N = 4, 20, 65536
K = 2048
CH = 8192
NV = CH // 16
NCHUNK = N // CH
UNROLL = 8
NKV = K // 16
NPAIR = B * C
NC, NS = 2, 16
NW = NC * NS
JMAX = (NPAIR + NW - 1) // NW


def _splat0(x):
    idx = jnp.zeros((16, 1), jnp.int32)
    return lax.gather(
        x, idx,
        lax.GatherDimensionNumbers(
            offset_dims=(), collapsed_slice_dims=(0,), start_index_map=(0,)),
        (1,), mode=lax.GatherScatterMode.PROMISE_IN_BOUNDS)


def _sc_body(probs_hbm, labels_hbm, res_hbm,
             pbuf0, pbuf1, lbig, hn, hf, stage, shn, shf, sem0, sem1, lsem):
    wid = lax.axis_index("s") * NC + lax.axis_index("c")
    zeros16 = jnp.zeros((16,), jnp.float32)
    ones16 = jnp.ones((16,), jnp.float32)
    iota_f = lax.iota(jnp.int32, 16).astype(jnp.float32)
    pbufs = (pbuf0, pbuf1)
    sems = (sem0, sem1)

    b = wid // 8
    s8 = wid % 8
    pltpu.async_copy(labels_hbm.at[b], lbig, lsem)

    def zero_body(i, _):
        hn[pl.ds(i * 16, 16)] = zeros16
        hf[pl.ds(i * 16, 16)] = zeros16
        return 0

    lax.fori_loop(0, NKV, zero_body, 0)
    pltpu.make_async_copy(labels_hbm.at[0], lbig, lsem).wait()

    def scatter_class(c, c0, base_off, nchunk):
        def start_load(ci, slot):
            pltpu.async_copy(probs_hbm.at[b, c, pl.ds(base_off + ci * CH, CH)],
                             pbufs[slot], sems[slot])

        def drain(slot):
            pltpu.make_async_copy(probs_hbm.at[0, 0, pl.ds(0, CH)],
                                  pbufs[slot], sems[slot]).wait()

        start_load(0, 0)
        for ci in range(nchunk):
            slot = ci % 2
            drain(slot)
            if ci + 1 < nchunk:
                start_load(ci + 1, 1 - slot)
            pbuf = pbufs[slot]
            lbase = base_off + ci * CH

            def vec_body(vi, _):
                base = vi * (16 * UNROLL)
                loaded = []
                for u in range(UNROLL):
                    o = base + u * 16
                    loaded.append((pbuf[pl.ds(o, 16)],
                                   lbig[pl.ds(lbase + o, 16)]))
                outs = []
                for pv, lv in loaded:
                    valid = lv != 0
                    isfg = lv == c0
                    e = jnp.where(isfg, 1.0 - pv, pv)
                    ec = jnp.minimum(e, 0.99999994)
                    bin_ = (ec * K).astype(jnp.int32)
                    outs.append((valid, isfg, bin_))
                for valid, isfg, bin_ in outs:
                    plsc.addupdate_scatter(hn, [bin_], ones16, mask=valid)
                    plsc.addupdate_scatter(hf, [bin_], ones16, mask=isfg)
                return 0

            lax.fori_loop(0, NV // UNROLL, vec_body, 0)

    def reduce_and_write(p):
        def psum_body(i, acc):
            return acc + hf[pl.ds(i * 16, 16)]

        P = jnp.sum(lax.fori_loop(0, NKV, psum_body, zeros16))

        def scan_body(i, carry):
            Tc, Fc, acc = carry
            kc = NKV - 1 - i
            nv = hn[pl.ds(kc * 16, 16)]
            fv = hf[pl.ds(kc * 16, 16)]
            hn[pl.ds(kc * 16, 16)] = zeros16
            hf[pl.ds(kc * 16, 16)] = zeros16
            Tin = jnp.flip(jnp.cumsum(jnp.flip(nv, 0)), 0) + Tc
            Fin = jnp.flip(jnp.cumsum(jnp.flip(fv, 0)), 0) + Fc
            Tex = Tin - nv
            Fex = Fin - fv
            din = jnp.maximum(P + (Tin - Fin), 1.0)
            dex = jnp.maximum(P + (Tex - Fex), 1.0)
            num = (P - Fex) * din - (P - Fin) * dex
            dj = num * (1.0 / (din * dex))
            vbar = (iota_f + (kc * 16 + 0.5).astype(jnp.float32)) * (1.0 / K)
            acc = acc + vbar * dj
            return (_splat0(Tin), _splat0(Fin), acc)

        _, _, acc = lax.fori_loop(0, NKV, scan_body,
                                  (zeros16, zeros16, zeros16))
        total_c = jnp.sum(acc)

        lane = lax.iota(jnp.int32, 16)
        row = jnp.where(lane == 0, total_c, jnp.where(lane == 1, P, 0.0))
        stage[...] = row.astype(jnp.float32)
        pltpu.sync_copy(stage, res_hbm.at[p])

    for j in range(2):
        c = s8 + 8 * j
        c0 = jnp.where(c == 0, jnp.int32(-1), c)
        scatter_class(c, c0, 0, NCHUNK)
        reduce_and_write(b * C + c)

    cs = 16 + (s8 % 4)
    is_a = s8 < 4
    half_off = jnp.where(is_a, 0, N // 2)
    scatter_class(cs, cs, half_off, NCHUNK // 2)
    slot = b * 4 + (s8 % 4)

    @pl.when(jnp.logical_not(is_a))
    def _():
        pltpu.sync_copy(hn, shn.at[slot])
        pltpu.sync_copy(hf, shf.at[slot])

    plsc.subcore_barrier()

    @pl.when(is_a)
    def _():
        pltpu.sync_copy(shn.at[slot], pbuf0.at[pl.ds(0, K)])
        pltpu.sync_copy(shf.at[slot], pbuf1.at[pl.ds(0, K)])

        def merge_body(i, _):
            ds = pl.ds(i * 16, 16)
            hn[ds] = hn[ds] + pbuf0[ds]
            hf[ds] = hf[ds] + pbuf1[ds]
            return 0

        lax.fori_loop(0, NKV, merge_body, 0)
        reduce_and_write(b * C + cs)


_sc_kernel = functools.partial(
    pl.kernel,
    mesh=plsc.VectorSubcoreMesh(core_axis_name="c", subcore_axis_name="s"),
    compiler_params=pltpu.CompilerParams(needs_layout_passes=False),
    out_type=jax.ShapeDtypeStruct((NPAIR, 16), jnp.float32),
    scratch_types=[
        pltpu.VMEM((CH,), jnp.float32),
        pltpu.VMEM((CH,), jnp.float32),
        pltpu.VMEM((N,), jnp.int32),
        pltpu.VMEM((K,), jnp.float32),
        pltpu.VMEM((K,), jnp.float32),
        pltpu.VMEM((16,), jnp.float32),
        pltpu.VMEM_SHARED((16, K), jnp.float32),
        pltpu.VMEM_SHARED((16, K), jnp.float32),
        pltpu.SemaphoreType.DMA,
        pltpu.SemaphoreType.DMA,
        pltpu.SemaphoreType.DMA,
    ],
)(_sc_body)


def _combine_body(tot_ref, p_ref, out_ref):
    totals = tot_ref[...]
    present = (p_ref[...] > 0).astype(jnp.float32)
    n = jnp.sum(present, axis=1, keepdims=True)
    tsum = jnp.sum(present * totals, axis=1, keepdims=True)
    loss_b = jnp.where(n > 0, tsum / jnp.maximum(n, 1.0), 0.0)
    out_ref[...] = jnp.sum(loss_b, keepdims=True) * (1.0 / B)


def kernel(uv_out, uv_label):
    res = _sc_kernel(uv_out, uv_label)
    totals = res[:, 0].reshape(B, C)
    ps = res[:, 1].reshape(B, C)
    out = pl.pallas_call(
        _combine_body,
        out_shape=jax.ShapeDtypeStruct((1, 1), jnp.float32),
    )(totals, ps)
    return out[0, 0]


# speedup vs baseline: 1.2453x; 1.0356x over previous
import functools

import jax
import jax.numpy as jnp
from jax import lax
from jax.experimental import pallas as pl
from jax.experimental.pallas import tpu as pltpu
from jax.experimental.pallas import tpu_sc as plsc

B, C, N = 4, 20, 65536
K = 1024
CH = 8192
NV = CH // 16
NCHUNK = N // CH
UNROLL = 8
NKV = K // 16
NPAIR = B * C
NC, NS = 2, 16
NW = NC * NS
JMAX = (NPAIR + NW - 1) // NW


def _splat0(x):
    idx = jnp.zeros((16, 1), jnp.int32)
    return lax.gather(
        x, idx,
        lax.GatherDimensionNumbers(
            offset_dims=(), collapsed_slice_dims=(0,), start_index_map=(0,)),
        (1,), mode=lax.GatherScatterMode.PROMISE_IN_BOUNDS)


def _sc_body(probs_hbm, labels_hbm, res_hbm,
             pbuf0, pbuf1, lbig, hn, hf, stage, shn, shf, sem0, sem1, lsem):
    wid = lax.axis_index("s") * NC + lax.axis_index("c")
    zeros16 = jnp.zeros((16,), jnp.float32)
    ones16 = jnp.ones((16,), jnp.float32)
    iota_f = lax.iota(jnp.int32, 16).astype(jnp.float32)
    pbufs = (pbuf0, pbuf1)
    sems = (sem0, sem1)

    b = wid // 8
    s8 = wid % 8
    pltpu.async_copy(labels_hbm.at[b], lbig, lsem)

    def zero_body(i, _):
        hn[pl.ds(i * 16, 16)] = zeros16
        hf[pl.ds(i * 16, 16)] = zeros16
        return 0

    lax.fori_loop(0, NKV, zero_body, 0)
    pltpu.make_async_copy(labels_hbm.at[0], lbig, lsem).wait()

    def scatter_class(c, c0, base_off, nchunk):
        def start_load(ci, slot):
            pltpu.async_copy(probs_hbm.at[b, c, pl.ds(base_off + ci * CH, CH)],
                             pbufs[slot], sems[slot])

        def drain(slot):
            pltpu.make_async_copy(probs_hbm.at[0, 0, pl.ds(0, CH)],
                                  pbufs[slot], sems[slot]).wait()

        start_load(0, 0)
        for ci in range(nchunk):
            slot = ci % 2
            drain(slot)
            if ci + 1 < nchunk:
                start_load(ci + 1, 1 - slot)
            pbuf = pbufs[slot]
            lbase = base_off + ci * CH

            def vec_body(vi, _):
                base = vi * (16 * UNROLL)
                loaded = []
                for u in range(UNROLL):
                    o = base + u * 16
                    loaded.append((pbuf[pl.ds(o, 16)],
                                   lbig[pl.ds(lbase + o, 16)]))
                outs = []
                for pv, lv in loaded:
                    valid = lv != 0
                    isfg = lv == c0
                    e = jnp.where(isfg, 1.0 - pv, pv)
                    ec = jnp.minimum(e, 0.99999994)
                    bin_ = (ec * K).astype(jnp.int32)
                    outs.append((valid, isfg, bin_))
                for valid, isfg, bin_ in outs:
                    plsc.addupdate_scatter(hn, [bin_], ones16, mask=valid)
                    plsc.addupdate_scatter(hf, [bin_], ones16, mask=isfg)
                return 0

            lax.fori_loop(0, NV // UNROLL, vec_body, 0)

    def reduce_and_write(p):
        def psum_body(i, acc):
            return acc + hf[pl.ds(i * 16, 16)]

        P = jnp.sum(lax.fori_loop(0, NKV, psum_body, zeros16))

        def scan_body(i, carry):
            Tc, Fc, acc = carry
            kc = NKV - 1 - i
            nv = hn[pl.ds(kc * 16, 16)]
            fv = hf[pl.ds(kc * 16, 16)]
            hn[pl.ds(kc * 16, 16)] = zeros16
            hf[pl.ds(kc * 16, 16)] = zeros16
            Tin = jnp.flip(jnp.cumsum(jnp.flip(nv, 0)), 0) + Tc
            Fin = jnp.flip(jnp.cumsum(jnp.flip(fv, 0)), 0) + Fc
            Tex = Tin - nv
            Fex = Fin - fv
            din = jnp.maximum(P + (Tin - Fin), 1.0)
            dex = jnp.maximum(P + (Tex - Fex), 1.0)
            num = (P - Fex) * din - (P - Fin) * dex
            dj = num * (1.0 / (din * dex))
            vbar = (iota_f + (kc * 16 + 0.5).astype(jnp.float32)) * (1.0 / K)
            acc = acc + vbar * dj
            return (_splat0(Tin), _splat0(Fin), acc)

        _, _, acc = lax.fori_loop(0, NKV, scan_body,
                                  (zeros16, zeros16, zeros16))
        total_c = jnp.sum(acc)

        lane = lax.iota(jnp.int32, 16)
        row = jnp.where(lane == 0, total_c, jnp.where(lane == 1, P, 0.0))
        stage[...] = row.astype(jnp.float32)
        pltpu.sync_copy(stage, res_hbm.at[p])

    for j in range(2):
        c = s8 + 8 * j
        c0 = jnp.where(c == 0, jnp.int32(-1), c)
        scatter_class(c, c0, 0, NCHUNK)
        reduce_and_write(b * C + c)

    cs = 16 + (s8 % 4)
    is_a = s8 < 4
    half_off = jnp.where(is_a, 0, N // 2)
    scatter_class(cs, cs, half_off, NCHUNK // 2)
    slot = b * 4 + (s8 % 4)

    @pl.when(jnp.logical_not(is_a))
    def _():
        pltpu.sync_copy(hn, shn.at[slot])
        pltpu.sync_copy(hf, shf.at[slot])

    plsc.subcore_barrier()

    @pl.when(is_a)
    def _():
        pltpu.sync_copy(shn.at[slot], pbuf0.at[pl.ds(0, K)])
        pltpu.sync_copy(shf.at[slot], pbuf1.at[pl.ds(0, K)])

        def merge_body(i, _):
            ds = pl.ds(i * 16, 16)
            hn[ds] = hn[ds] + pbuf0[ds]
            hf[ds] = hf[ds] + pbuf1[ds]
            return 0

        lax.fori_loop(0, NKV, merge_body, 0)
        reduce_and_write(b * C + cs)


_sc_kernel = functools.partial(
    pl.kernel,
    mesh=plsc.VectorSubcoreMesh(core_axis_name="c", subcore_axis_name="s"),
    compiler_params=pltpu.CompilerParams(needs_layout_passes=False),
    out_type=jax.ShapeDtypeStruct((NPAIR, 16), jnp.float32),
    scratch_types=[
        pltpu.VMEM((CH,), jnp.float32),
        pltpu.VMEM((CH,), jnp.float32),
        pltpu.VMEM((N,), jnp.int32),
        pltpu.VMEM((K,), jnp.float32),
        pltpu.VMEM((K,), jnp.float32),
        pltpu.VMEM((16,), jnp.float32),
        pltpu.VMEM_SHARED((16, K), jnp.float32),
        pltpu.VMEM_SHARED((16, K), jnp.float32),
        pltpu.SemaphoreType.DMA,
        pltpu.SemaphoreType.DMA,
        pltpu.SemaphoreType.DMA,
    ],
)(_sc_body)


def _combine_body(res_ref, out_ref):
    x = res_ref[...]
    totals = x[:, :, 0]
    present = (x[:, :, 1] > 0).astype(jnp.float32)
    n = jnp.sum(present, axis=1, keepdims=True)
    tsum = jnp.sum(present * totals, axis=1, keepdims=True)
    loss_b = jnp.where(n > 0, tsum / jnp.maximum(n, 1.0), 0.0)
    out_ref[...] = jnp.sum(loss_b, keepdims=True) * (1.0 / B)


def kernel(uv_out, uv_label):
    res = _sc_kernel(uv_out, uv_label)
    out = pl.pallas_call(
        _combine_body,
        out_shape=jax.ShapeDtypeStruct((1, 1), jnp.float32),
    )(res.reshape(B, C, 16))
    return out[0, 0]
